# baseline (speedup 1.0000x reference)
import functools

import jax
import jax.numpy as jnp
from jax import lax
from jax.experimental import pallas as pl
from jax.experimental.pallas import tpu as pltpu
from jax.experimental.pallas import tpu_sc as plsc

TB = 64
VP = 128


def _sc_gather(table, idx):
    n, d = idx.shape[0], table.shape[1]
    info = plsc.get_sparse_core_info()
    nw = info.num_cores * info.num_subcores
    b_per_w = n // nw
    ch = 256
    mesh = plsc.VectorSubcoreMesh(core_axis_name="c", subcore_axis_name="s")

    @functools.partial(
        pl.kernel, mesh=mesh,
        out_type=jax.ShapeDtypeStruct((n, d), jnp.float32),
        scratch_types=[
            pltpu.VMEM((ch,), jnp.int32),
            pltpu.VMEM((ch, d), jnp.float32),
            pltpu.SemaphoreType.DMA,
        ],
    )
    def gather_kernel(table_hbm, idx_hbm, out_hbm, idx_v, rows_v, sem):
        wid = lax.axis_index("s") * info.num_cores + lax.axis_index("c")
        base = wid * b_per_w

        def chunk(j, carry):
            off = base + j * ch
            pltpu.sync_copy(idx_hbm.at[pl.ds(off, ch)], idx_v)
            pltpu.async_copy(table_hbm.at[idx_v], rows_v, sem).wait()
            pltpu.sync_copy(rows_v, out_hbm.at[pl.ds(off, ch)])
            return carry

        lax.fori_loop(0, b_per_w // ch, chunk, 0)

    return gather_kernel(table, idx)


def _selu(x):
    alpha = 1.6732632423543772848170429916717
    scale = 1.0507009873554804934193349852946
    return scale * jnp.where(x > 0, x, alpha * (jnp.exp(jnp.minimum(x, 0.0)) - 1.0))


def _shift_rows(x, k):
    if k == 0:
        return x
    return jnp.concatenate([x[k:], x[:k]], axis=0)


def _branch(emb_ref, w0_ref, b0_ref, w1_ref, b1_ref, L, K):
    n = emb_ref.shape[0]
    tb = n // L
    f1 = w1_ref.shape[1]
    emb = emb_ref[:, :w0_ref.shape[0] // K].astype(jnp.bfloat16)
    x0 = jnp.concatenate([_shift_rows(emb, k) for k in range(K)], axis=1)
    acc0 = jnp.dot(x0, w0_ref[...], preferred_element_type=jnp.float32)
    y0 = _selu(acc0 + b0_ref[...]).astype(jnp.bfloat16)
    x1 = jnp.concatenate([_shift_rows(y0, k) for k in range(K)], axis=1)
    acc1 = jnp.dot(x1, w1_ref[...], preferred_element_type=jnp.float32)
    valid = L - 2 * (K - 1)
    m = jnp.max(acc1.reshape(tb, L, f1)[:, :valid, :], axis=1)
    return _selu(m + b1_ref[...])


def _body(emba_ref, embc_ref,
          wa0_ref, ba0_ref, wa1_ref, ba1_ref,
          wc0_ref, bc0_ref, wc1_ref, bc1_ref,
          wd0_ref, bd0_ref, wd1_ref, bd1_ref, wd2_ref, bd2_ref,
          wh_ref, bh_ref, out_ref, h_ref, *, L, K, nsteps):
    i = pl.program_id(0)
    a = _branch(emba_ref, wa0_ref, ba0_ref, wa1_ref, ba1_ref, L, K)
    c = _branch(embc_ref, wc0_ref, bc0_ref, wc1_ref, bc1_ref, L, K)
    f1 = a.shape[1]
    h_ref[pl.ds(i * a.shape[0], a.shape[0]), :f1] = a.astype(jnp.bfloat16)
    h_ref[pl.ds(i * a.shape[0], a.shape[0]), f1:] = c.astype(jnp.bfloat16)

    @pl.when(i == nsteps - 1)
    def _mlp():
        h = h_ref[...]
        h = jax.nn.relu(jnp.dot(h, wd0_ref[...],
                                preferred_element_type=jnp.float32)
                        + bd0_ref[...]).astype(jnp.bfloat16)
        h = jax.nn.relu(jnp.dot(h, wd1_ref[...],
                                preferred_element_type=jnp.float32)
                        + bd1_ref[...]).astype(jnp.bfloat16)
        h = jax.nn.relu(jnp.dot(h, wd2_ref[...],
                                preferred_element_type=jnp.float32)
                        + bd2_ref[...]).astype(jnp.bfloat16)
        o = (jnp.dot(h, wh_ref[...], preferred_element_type=jnp.float32)
             + bh_ref[...])
        out_ref[...] = jax.nn.sigmoid(o)


def kernel(api, cof, E_api, E_cof, Wa0, ba0, Wa1, ba1, Wc0, bc0, Wc1, bc1,
           Wd0, bd0, Wd1, bd1, Wd2, bd2, Wh, bh):
    B, L = api.shape
    V, D = E_api.shape
    K = Wa0.shape[0]
    F1 = Wa1.shape[2]

    ea = jnp.zeros((V, VP), jnp.float32).at[:, :D].set(E_api)
    ec = jnp.zeros((V, VP), jnp.float32).at[:, :D].set(E_cof)
    emb_a = _sc_gather(ea, api.astype(jnp.int32).reshape(B * L))
    emb_c = _sc_gather(ec, cof.astype(jnp.int32).reshape(B * L))
    bf = lambda a: a.astype(jnp.bfloat16)

    full = lambda arr: pl.BlockSpec(arr.shape, lambda i: (0,) * arr.ndim)
    row2 = lambda a: a.reshape(1, -1)

    args = (emb_a, emb_c,
            bf(Wa0.reshape(K * D, Wa0.shape[2])), row2(ba0),
            bf(Wa1.reshape(K * Wa1.shape[1], F1)), row2(ba1),
            bf(Wc0.reshape(K * D, Wc0.shape[2])), row2(bc0),
            bf(Wc1.reshape(K * Wc1.shape[1], F1)), row2(bc1),
            bf(Wd0), row2(bd0), bf(Wd1), row2(bd1), bf(Wd2), row2(bd2),
            bf(Wh), row2(bh))
    in_specs = [pl.BlockSpec((TB * L, VP), lambda i: (i, 0)),
                pl.BlockSpec((TB * L, VP), lambda i: (i, 0))]
    in_specs += [full(a) for a in args[2:]]

    return pl.pallas_call(
        functools.partial(_body, L=L, K=K, nsteps=B // TB),
        grid=(B // TB,),
        in_specs=in_specs,
        out_specs=pl.BlockSpec((B, 1), lambda i: (0, 0)),
        out_shape=jax.ShapeDtypeStruct((B, 1), jnp.float32),
        scratch_shapes=[pltpu.VMEM((B, 2 * F1), jnp.bfloat16)],
    )(*args)


# speedup vs baseline: 2.0289x; 2.0289x over previous
import functools

import jax
import jax.numpy as jnp
from jax.experimental import pallas as pl
from jax.experimental.pallas import tpu as pltpu

TB = 64
VP = 128


def _selu(x):
    alpha = 1.6732632423543772848170429916717
    scale = 1.0507009873554804934193349852946
    return scale * jnp.where(x > 0, x, alpha * (jnp.exp(jnp.minimum(x, 0.0)) - 1.0))


def _shift_rows(x, k):
    if k == 0:
        return x
    return jnp.concatenate([x[k:], x[:k]], axis=0)


def _branch(ids, e_ref, w0_ref, b0_ref, w1_ref, b1_ref, L, K):
    tb = ids.shape[0]
    n = tb * L
    f1 = w1_ref.shape[1]
    iota3 = jax.lax.broadcasted_iota(jnp.int32, (tb, L, VP), 2)
    oh = (ids[:, :, None] == iota3).astype(jnp.bfloat16).reshape(n, VP)
    emb = jnp.dot(oh, e_ref[...],
                  preferred_element_type=jnp.float32).astype(jnp.bfloat16)
    x0 = jnp.concatenate([_shift_rows(emb, k) for k in range(K)], axis=1)
    acc0 = jnp.dot(x0, w0_ref[...], preferred_element_type=jnp.float32)
    y0 = _selu(acc0 + b0_ref[...]).astype(jnp.bfloat16)
    x1 = jnp.concatenate([_shift_rows(y0, k) for k in range(K)], axis=1)
    acc1 = jnp.dot(x1, w1_ref[...], preferred_element_type=jnp.float32)
    valid = L - 2 * (K - 1)
    m = jnp.max(acc1.reshape(tb, L, f1)[:, :valid, :], axis=1)
    return _selu(m + b1_ref[...])


def _body(api_ref, cof_ref, ea_ref, ec_ref,
          wa0_ref, ba0_ref, wa1_ref, ba1_ref,
          wc0_ref, bc0_ref, wc1_ref, bc1_ref,
          wd0_ref, bd0_ref, wd1_ref, bd1_ref, wd2_ref, bd2_ref,
          wh_ref, bh_ref, out_ref, h_ref, *, L, K, nsteps):
    i = pl.program_id(0)
    a = _branch(api_ref[...], ea_ref, wa0_ref, ba0_ref, wa1_ref, ba1_ref, L, K)
    c = _branch(cof_ref[...], ec_ref, wc0_ref, bc0_ref, wc1_ref, bc1_ref, L, K)
    f1 = a.shape[1]
    h_ref[pl.ds(i * a.shape[0], a.shape[0]), :f1] = a.astype(jnp.bfloat16)
    h_ref[pl.ds(i * a.shape[0], a.shape[0]), f1:] = c.astype(jnp.bfloat16)

    @pl.when(i == nsteps - 1)
    def _mlp():
        h = h_ref[...]
        h = jax.nn.relu(jnp.dot(h, wd0_ref[...],
                                preferred_element_type=jnp.float32)
                        + bd0_ref[...]).astype(jnp.bfloat16)
        h = jax.nn.relu(jnp.dot(h, wd1_ref[...],
                                preferred_element_type=jnp.float32)
                        + bd1_ref[...]).astype(jnp.bfloat16)
        h = jax.nn.relu(jnp.dot(h, wd2_ref[...],
                                preferred_element_type=jnp.float32)
                        + bd2_ref[...]).astype(jnp.bfloat16)
        o = (jnp.dot(h, wh_ref[...], preferred_element_type=jnp.float32)
             + bh_ref[...])
        out_ref[...] = jax.nn.sigmoid(o)


def kernel(api, cof, E_api, E_cof, Wa0, ba0, Wa1, ba1, Wc0, bc0, Wc1, bc1,
           Wd0, bd0, Wd1, bd1, Wd2, bd2, Wh, bh):
    B, L = api.shape
    V, D = E_api.shape
    K = Wa0.shape[0]
    F1 = Wa1.shape[2]

    api32 = api.astype(jnp.int32)
    cof32 = cof.astype(jnp.int32)
    bf = lambda a: a.astype(jnp.bfloat16)
    ea = bf(jnp.zeros((VP, D), jnp.float32).at[:V].set(E_api))
    ec = bf(jnp.zeros((VP, D), jnp.float32).at[:V].set(E_cof))

    full = lambda arr: pl.BlockSpec(arr.shape, lambda i: (0,) * arr.ndim)
    row2 = lambda a: a.reshape(1, -1)

    args = (api32, cof32, ea, ec,
            bf(Wa0.reshape(K * D, Wa0.shape[2])), row2(ba0),
            bf(Wa1.reshape(K * Wa1.shape[1], F1)), row2(ba1),
            bf(Wc0.reshape(K * D, Wc0.shape[2])), row2(bc0),
            bf(Wc1.reshape(K * Wc1.shape[1], F1)), row2(bc1),
            bf(Wd0), row2(bd0), bf(Wd1), row2(bd1), bf(Wd2), row2(bd2),
            bf(Wh), row2(bh))
    in_specs = [pl.BlockSpec((TB, L), lambda i: (i, 0)),
                pl.BlockSpec((TB, L), lambda i: (i, 0))]
    in_specs += [full(a) for a in args[2:]]

    return pl.pallas_call(
        functools.partial(_body, L=L, K=K, nsteps=B // TB),
        grid=(B // TB,),
        in_specs=in_specs,
        out_specs=pl.BlockSpec((B, 1), lambda i: (0, 0)),
        out_shape=jax.ShapeDtypeStruct((B, 1), jnp.float32),
        scratch_shapes=[pltpu.VMEM((B, 2 * F1), jnp.bfloat16)],
    )(*args)


# speedup vs baseline: 2.0361x; 1.0035x over previous
import functools

import jax
import jax.numpy as jnp
from jax.experimental import pallas as pl
from jax.experimental.pallas import tpu as pltpu

TB = 64
VP = 128


def _selu(x):
    alpha = 1.6732632423543772848170429916717
    scale = 1.0507009873554804934193349852946
    return (scale * jnp.maximum(x, 0.0)
            + (scale * alpha) * (jnp.exp(jnp.minimum(x, 0.0)) - 1.0))


def _shift_rows(x, k):
    if k == 0:
        return x
    return jnp.concatenate([x[k:], x[:k]], axis=0)


def _branch(ids, e_ref, w0_ref, b0_ref, w1_ref, b1_ref, L, K):
    tb = ids.shape[0]
    n = tb * L
    f1 = w1_ref.shape[1]
    iota3 = jax.lax.broadcasted_iota(jnp.int32, (tb, L, VP), 2)
    oh = (ids[:, :, None] == iota3).astype(jnp.bfloat16).reshape(n, VP)
    emb = jnp.dot(oh, e_ref[...],
                  preferred_element_type=jnp.float32).astype(jnp.bfloat16)
    x0 = jnp.concatenate([_shift_rows(emb, k) for k in range(K)], axis=1)
    acc0 = jnp.dot(x0, w0_ref[...], preferred_element_type=jnp.float32)
    y0 = _selu(acc0 + b0_ref[...]).astype(jnp.bfloat16)
    x1 = jnp.concatenate([_shift_rows(y0, k) for k in range(K)], axis=1)
    acc1 = jnp.dot(x1, w1_ref[...], preferred_element_type=jnp.float32)
    valid = L - 2 * (K - 1)
    m = jnp.max(acc1.reshape(tb, L, f1)[:, :valid, :], axis=1)
    return _selu(m + b1_ref[...])


def _body(api_ref, cof_ref, ea_ref, ec_ref,
          wa0_ref, ba0_ref, wa1_ref, ba1_ref,
          wc0_ref, bc0_ref, wc1_ref, bc1_ref,
          wd0_ref, bd0_ref, wd1_ref, bd1_ref, wd2_ref, bd2_ref,
          wh_ref, bh_ref, out_ref, h_ref, *, L, K, nsteps):
    i = pl.program_id(0)
    a = _branch(api_ref[...], ea_ref, wa0_ref, ba0_ref, wa1_ref, ba1_ref, L, K)
    c = _branch(cof_ref[...], ec_ref, wc0_ref, bc0_ref, wc1_ref, bc1_ref, L, K)
    f1 = a.shape[1]
    h_ref[pl.ds(i * a.shape[0], a.shape[0]), :f1] = a.astype(jnp.bfloat16)
    h_ref[pl.ds(i * a.shape[0], a.shape[0]), f1:] = c.astype(jnp.bfloat16)

    @pl.when(i == nsteps - 1)
    def _mlp():
        h = h_ref[...]
        h = jax.nn.relu(jnp.dot(h, wd0_ref[...],
                                preferred_element_type=jnp.float32)
                        + bd0_ref[...]).astype(jnp.bfloat16)
        h = jax.nn.relu(jnp.dot(h, wd1_ref[...],
                                preferred_element_type=jnp.float32)
                        + bd1_ref[...]).astype(jnp.bfloat16)
        h = jax.nn.relu(jnp.dot(h, wd2_ref[...],
                                preferred_element_type=jnp.float32)
                        + bd2_ref[...]).astype(jnp.bfloat16)
        o = (jnp.dot(h, wh_ref[...], preferred_element_type=jnp.float32)
             + bh_ref[...])
        out_ref[...] = jax.nn.sigmoid(o)


def kernel(api, cof, E_api, E_cof, Wa0, ba0, Wa1, ba1, Wc0, bc0, Wc1, bc1,
           Wd0, bd0, Wd1, bd1, Wd2, bd2, Wh, bh):
    B, L = api.shape
    V, D = E_api.shape
    K = Wa0.shape[0]
    F1 = Wa1.shape[2]

    api32 = api.astype(jnp.int32)
    cof32 = cof.astype(jnp.int32)
    bf = lambda a: a.astype(jnp.bfloat16)
    ea = bf(jnp.zeros((VP, D), jnp.float32).at[:V].set(E_api))
    ec = bf(jnp.zeros((VP, D), jnp.float32).at[:V].set(E_cof))

    full = lambda arr: pl.BlockSpec(arr.shape, lambda i: (0,) * arr.ndim)
    row2 = lambda a: a.reshape(1, -1)

    args = (api32, cof32, ea, ec,
            bf(Wa0.reshape(K * D, Wa0.shape[2])), row2(ba0),
            bf(Wa1.reshape(K * Wa1.shape[1], F1)), row2(ba1),
            bf(Wc0.reshape(K * D, Wc0.shape[2])), row2(bc0),
            bf(Wc1.reshape(K * Wc1.shape[1], F1)), row2(bc1),
            bf(Wd0), row2(bd0), bf(Wd1), row2(bd1), bf(Wd2), row2(bd2),
            bf(Wh), row2(bh))
    in_specs = [pl.BlockSpec((TB, L), lambda i: (i, 0)),
                pl.BlockSpec((TB, L), lambda i: (i, 0))]
    in_specs += [full(a) for a in args[2:]]

    return pl.pallas_call(
        functools.partial(_body, L=L, K=K, nsteps=B // TB),
        grid=(B // TB,),
        in_specs=in_specs,
        out_specs=pl.BlockSpec((B, 1), lambda i: (0, 0)),
        out_shape=jax.ShapeDtypeStruct((B, 1), jnp.float32),
        scratch_shapes=[pltpu.VMEM((B, 2 * F1), jnp.bfloat16)],
    )(*args)
